# baseline (speedup 1.0000x reference)
import functools

import jax
import jax.numpy as jnp
from jax import lax
from jax.experimental import pallas as pl
from jax.experimental.pallas import tpu as pltpu
from jax.experimental.pallas import tpu_sc as plsc

_NC = 2
_NS = 16
_NW = _NC * _NS

_TR = 16
_LANES = 16
_UNROLL = 4


def _sc_add(B, seq_len, D):
    rows_per_w = seq_len // _NW
    steps = rows_per_w // _TR
    tile = _TR * D
    n_iter = tile // (_LANES * _UNROLL)

    mesh = plsc.VectorSubcoreMesh(core_axis_name="c", subcore_axis_name="s")

    @functools.partial(
        pl.kernel,
        mesh=mesh,
        out_type=jax.ShapeDtypeStruct((B, seq_len * D), jnp.float32),
        scratch_types=(
            [pltpu.VMEM((tile,), jnp.float32)]
            + [pltpu.VMEM((tile,), jnp.float32) for _ in range(B)]
        ),
    )
    def run(x_hbm, pe_hbm, out_hbm, pe_v, *x_vs):
        wid = lax.axis_index("s") * _NC + lax.axis_index("c")
        base = wid * rows_per_w * D

        def step(t, carry):
            elem0 = base + t * tile
            pltpu.sync_copy(pe_hbm.at[pl.ds(elem0, tile)], pe_v)
            for b in range(B):
                pltpu.sync_copy(x_hbm.at[b, pl.ds(elem0, tile)], x_vs[b])

            def add_body(j, c):
                for u in range(_UNROLL):
                    off = (j * _UNROLL + u) * _LANES
                    sl = pl.ds(off, _LANES)
                    pv = pe_v[sl]
                    for b in range(B):
                        x_vs[b][sl] = x_vs[b][sl] + pv
                return c

            lax.fori_loop(0, n_iter, add_body, 0, unroll=False)

            for b in range(B):
                pltpu.sync_copy(x_vs[b], out_hbm.at[b, pl.ds(elem0, tile)])
            return carry

        lax.fori_loop(0, steps, step, 0, unroll=False)

    return run


def kernel(x, pe_table):
    B, S, D = x.shape
    seq_len = min(S, pe_table.shape[0])
    xf = x[:, :seq_len, :].reshape(B, seq_len * D)
    pf = pe_table[:seq_len].reshape(seq_len * D)
    out = _sc_add(B, seq_len, D)(xf, pf)
    return out.reshape(B, seq_len, D)


# speedup vs baseline: 2.0283x; 2.0283x over previous
import functools

import jax
import jax.numpy as jnp
from jax import lax
from jax.experimental import pallas as pl
from jax.experimental.pallas import tpu as pltpu
from jax.experimental.pallas import tpu_sc as plsc

_NC = 2
_NS = 16
_NW = _NC * _NS

_TR = 16
_LANES = 16
_UNROLL = 4


def _sc_add(B, seq_len, D):
    rows_per_w = seq_len // _NW
    steps = rows_per_w // _TR
    tile = _TR * D
    n_iter = tile // (_LANES * _UNROLL)

    mesh = plsc.VectorSubcoreMesh(core_axis_name="c", subcore_axis_name="s")

    @functools.partial(
        pl.kernel,
        mesh=mesh,
        out_type=jax.ShapeDtypeStruct((B, seq_len, D), jnp.float32),
        scratch_types=(
            [pltpu.VMEM((_TR, D), jnp.float32)]
            + [pltpu.VMEM((_TR, D), jnp.float32) for _ in range(B)]
        ),
        compiler_params=pltpu.CompilerParams(use_tc_tiling_on_sc=True),
    )
    def run(x_hbm, pe_hbm, out_hbm, pe_v, *x_vs):
        wid = lax.axis_index("s") * _NC + lax.axis_index("c")
        base = wid * rows_per_w

        def step(t, carry):
            row0 = base + t * _TR
            pltpu.sync_copy(pe_hbm.at[pl.ds(row0, _TR)], pe_v)
            for b in range(B):
                pltpu.sync_copy(x_hbm.at[b, pl.ds(row0, _TR)], x_vs[b])

            def add_body(j, c):
                for u in range(_UNROLL):
                    flat = (j * _UNROLL + u) * _LANES
                    r = flat // D
                    sl = pl.ds(flat % D, _LANES)
                    pv = pe_v[r, sl]
                    for b in range(B):
                        x_vs[b][r, sl] = x_vs[b][r, sl] + pv
                return c

            lax.fori_loop(0, n_iter, add_body, 0, unroll=False)

            for b in range(B):
                pltpu.sync_copy(x_vs[b], out_hbm.at[b, pl.ds(row0, _TR)])
            return carry

        lax.fori_loop(0, steps, step, 0, unroll=False)

    return run


def kernel(x, pe_table):
    B, S, D = x.shape
    seq_len = min(S, pe_table.shape[0])
    return _sc_add(B, seq_len, D)(x[:, :seq_len, :], pe_table[:seq_len])


# speedup vs baseline: 3.4302x; 1.6912x over previous
import functools

import jax
import jax.numpy as jnp
from jax import lax
from jax.experimental import pallas as pl
from jax.experimental.pallas import tpu as pltpu
from jax.experimental.pallas import tpu_sc as plsc

_NC = 2
_NS = 16
_NW = _NC * _NS

_TR = 8
_LANES = 16
_UNROLL = 2


def _sc_add(B, seq_len, D):
    rows_per_w = seq_len // _NW
    steps = rows_per_w // _TR
    tile = _TR * D
    n_iter = tile // (_LANES * _UNROLL)

    mesh = plsc.VectorSubcoreMesh(core_axis_name="c", subcore_axis_name="s")

    scratch = (
        [pltpu.VMEM((_TR, D), jnp.float32) for _ in range(2)]
        + [pltpu.VMEM((_TR, D), jnp.float32) for _ in range(2 * B)]
        + [pltpu.SemaphoreType.DMA for _ in range(2)]
        + [pltpu.SemaphoreType.DMA for _ in range(2 * B)]
        + [pltpu.SemaphoreType.DMA for _ in range(2 * B)]
    )

    @functools.partial(
        pl.kernel,
        mesh=mesh,
        out_type=jax.ShapeDtypeStruct((B, seq_len, D), jnp.float32),
        scratch_types=scratch,
        compiler_params=pltpu.CompilerParams(use_tc_tiling_on_sc=True),
    )
    def run(x_hbm, pe_hbm, out_hbm, *s):
        pe_v = (s[0], s[1])
        x_v = (s[2:2 + B], s[2 + B:2 + 2 * B])
        pe_sem = (s[2 + 2 * B], s[3 + 2 * B])
        ld_sem = (s[4 + 2 * B:4 + 3 * B], s[4 + 3 * B:4 + 4 * B])
        st_sem = (s[4 + 4 * B:4 + 5 * B], s[4 + 5 * B:4 + 6 * B])

        wid = lax.axis_index("s") * _NC + lax.axis_index("c")
        base = wid * rows_per_w

        def pe_copy(t, q):
            return pltpu.make_async_copy(
                pe_hbm.at[pl.ds(base + t * _TR, _TR)], pe_v[q], pe_sem[q])

        def ld_copy(t, q, b):
            return pltpu.make_async_copy(
                x_hbm.at[b, pl.ds(base + t * _TR, _TR)], x_v[q][b],
                ld_sem[q][b])

        def st_copy(t, q, b):
            return pltpu.make_async_copy(
                x_v[q][b], out_hbm.at[b, pl.ds(base + t * _TR, _TR)],
                st_sem[q][b])

        pe_copy(0, 0).start()
        for b in range(B):
            ld_copy(0, 0, b).start()

        def halfstep(t, q):
            @pl.when(t + 1 < steps)
            def _():
                pe_copy(t + 1, 1 - q).start()

            for b in range(B):
                @pl.when(t + 1 < steps)
                def _():
                    @pl.when(t >= 1)
                    def _():
                        st_copy(t - 1, 1 - q, b).wait()
                    ld_copy(t + 1, 1 - q, b).start()

            pe_copy(t, q).wait()
            for b in range(B):
                ld_copy(t, q, b).wait()

            def add_body(j, c):
                for u in range(_UNROLL):
                    flat = (j * _UNROLL + u) * _LANES
                    r = flat // D
                    sl = pl.ds(flat % D, _LANES)
                    pv = pe_v[q][r, sl]
                    for b in range(B):
                        x_v[q][b][r, sl] = x_v[q][b][r, sl] + pv
                return c

            lax.fori_loop(0, n_iter, add_body, 0, unroll=False)

            for b in range(B):
                st_copy(t, q, b).start()

        def outer(tt, carry):
            halfstep(2 * tt, 0)
            halfstep(2 * tt + 1, 1)
            return carry

        lax.fori_loop(0, steps // 2, outer, 0, unroll=False)

        for q in range(2):
            for b in range(B):
                st_copy(steps - 2 + q, q, b).wait()

    return run


def kernel(x, pe_table):
    B, S, D = x.shape
    seq_len = min(S, pe_table.shape[0])
    return _sc_add(B, seq_len, D)(x[:, :seq_len, :], pe_table[:seq_len])
